# baseline (speedup 1.0000x reference)
import functools

import jax
import jax.numpy as jnp
from jax import lax
from jax.experimental import pallas as pl
from jax.experimental.pallas import tpu as pltpu
from jax.experimental.pallas import tpu_sc as plsc

_NC = 2
_NS = 16
_NW = _NC * _NS
_L = 16

_R = 128
_C = 8192
_K = 3
_R_SC = 32
_R_TC = _R - _R_SC
_NQ = 4
_QC = _C // _NQ
_NVQ = _QC // _L
_OPAD = 16


def _insert(state, v, iv):
  m1, m2, m3, i1, i2, i3 = state
  c1 = v > m1
  c2 = v > m2
  c3 = v > m3
  t1 = jnp.minimum(m1, v)
  n1 = jnp.maximum(m1, v)
  t2 = jnp.minimum(m2, t1)
  n2 = jnp.maximum(m2, t1)
  n3 = jnp.maximum(m3, t2)
  j1 = jnp.where(c1, iv, i1)
  j2 = jnp.where(c1, i1, jnp.where(c2, iv, i2))
  j3 = jnp.where(c2, i2, jnp.where(c3, iv, i3))
  return n1, n2, n3, j1, j2, j3


def _bcast_max(x):
  return plsc.cummax(jnp.flip(plsc.cummax(x)))


def _bcast_min_i32(x):
  return -_bcast_max(-x)


def _sc_body(x_hbm, out_hbm, xin, vout):
  w = lax.axis_index("s") * _NC + lax.axis_index("c")
  pltpu.sync_copy(x_hbm.at[pl.ds(w, 1)], xin)

  neg_inf = jnp.full((_L,), -jnp.inf, jnp.float32)
  zeros_i = jnp.zeros((_L,), jnp.int32)
  iota = lax.iota(jnp.int32, _L)
  big = jnp.full((_L,), jnp.int32(2**30))

  def body(t, carry):
    new = []
    iv = carry[-1]
    for q in range(_NQ):
      v = xin[0, pl.ds(q * _QC + t * _L, _L)]
      new.append(_insert(carry[q], v, iv + q * _QC))
    return (*new, iv + _L)

  init = tuple((neg_inf, neg_inf, neg_inf, zeros_i, zeros_i, zeros_i)
               for _ in range(_NQ))
  carry = lax.fori_loop(0, _NVQ, body, (*init, iota), unroll=2)

  st = carry[0]
  for q in range(1, _NQ):
    m1, m2, m3, i1, i2, i3 = carry[q]
    st = _insert(st, m1, i1)
    st = _insert(st, m2, i2)
    st = _insert(st, m3, i3)
  m1, m2, m3, i1, i2, i3 = st

  vacc = neg_inf
  iacc = zeros_i
  for k in range(_K):
    s = _bcast_max(m1)
    cand = _bcast_min_i32(jnp.where(m1 == s, i1, big))
    vacc = jnp.where(iota == k, s, vacc)
    iacc = jnp.where(iota == k, cand, iacc)
    hit = (m1 == s) & (i1 == cand)
    m1 = jnp.where(hit, m2, m1)
    i1 = jnp.where(hit, i2, i1)
    m2 = jnp.where(hit, m3, m2)
    i2 = jnp.where(hit, i3, i2)
    m3 = jnp.where(hit, neg_inf, m3)
  vout[0, 0, :] = vacc
  vout[1, 0, :] = plsc.bitcast(iacc, jnp.float32)

  pltpu.sync_copy(vout.at[0], out_hbm.at[0, pl.ds(w, 1)])
  pltpu.sync_copy(vout.at[1], out_hbm.at[1, pl.ds(w, 1)])


_topk_sc = functools.partial(
    pl.kernel,
    out_type=jax.ShapeDtypeStruct((2, _R_SC, _OPAD), jnp.float32),
    mesh=plsc.VectorSubcoreMesh(
        core_axis_name="c", subcore_axis_name="s",
        num_cores=_NC, num_subcores=_NS),
    compiler_params=pltpu.CompilerParams(needs_layout_passes=False),
    scratch_types=[
        pltpu.VMEM((1, _C), jnp.float32),
        pltpu.VMEM((2, 1, _OPAD), jnp.float32),
    ],
)(_sc_body)


_TB = 8
_TL = 128
_NVT = _C // _TL


_TG = 4
_NVG = _NVT // _TG


def _tc_kernel(x_ref, v_ref, i_ref):
  neg_inf = jnp.full((_TB, _TL), -jnp.inf, jnp.float32)
  zeros_i = jnp.zeros((_TB, _TL), jnp.int32)
  lane = lax.broadcasted_iota(jnp.int32, (_TB, _TL), 1)
  big = jnp.full((_TB, _TL), jnp.int32(2**30))

  def body(t, carry):
    new = []
    for g in range(_TG):
      c = g * _NVG + t
      v = x_ref[:, pl.ds(c * _TL, _TL)]
      new.append(_insert(carry[g], v, lane + c * _TL))
    return tuple(new)

  init = tuple((neg_inf, neg_inf, neg_inf, zeros_i, zeros_i, zeros_i)
               for _ in range(_TG))
  carry = lax.fori_loop(0, _NVG, body, init, unroll=2)

  st = carry[0]
  for g in range(1, _TG):
    m1, m2, m3, i1, i2, i3 = carry[g]
    st = _insert(st, m1, i1)
    st = _insert(st, m2, i2)
    st = _insert(st, m3, i3)
  m1, m2, m3, i1, i2, i3 = st

  vcols = []
  icols = []
  for _ in range(_K):
    s = jnp.max(m1, axis=1, keepdims=True)
    cand = jnp.min(jnp.where(m1 == s, i1, big), axis=1, keepdims=True)
    vcols.append(s)
    icols.append(cand)
    hit = (m1 == s) & (i1 == cand)
    m1 = jnp.where(hit, m2, m1)
    i1 = jnp.where(hit, i2, i1)
    m2 = jnp.where(hit, m3, m2)
    i2 = jnp.where(hit, i3, i2)
    m3 = jnp.where(hit, neg_inf, m3)
  v_ref[...] = jnp.concatenate(vcols, axis=1)
  i_ref[...] = jnp.concatenate(icols, axis=1)


_topk_tc = pl.pallas_call(
    _tc_kernel,
    grid=(_R_TC // _TB,),
    in_specs=[pl.BlockSpec((_TB, _C), lambda i: (i + _R_SC // _TB, 0))],
    out_specs=[
        pl.BlockSpec((_TB, _K), lambda i: (i, 0)),
        pl.BlockSpec((_TB, _K), lambda i: (i, 0)),
    ],
    out_shape=[
        jax.ShapeDtypeStruct((_R_TC, _K), jnp.float32),
        jax.ShapeDtypeStruct((_R_TC, _K), jnp.int32),
    ],
)


@jax.jit
def kernel(x):
  sc_out = _topk_sc(x)
  tc_vals, tc_idxs = _topk_tc(x)
  sc_sliced = sc_out[:, :, :_K]
  vals = jnp.concatenate([sc_sliced[0], tc_vals], axis=0)
  idxs = jnp.concatenate(
      [lax.bitcast_convert_type(sc_sliced[1], jnp.int32), tc_idxs], axis=0)
  return vals, idxs


# speedup vs baseline: 1.0306x; 1.0306x over previous
import functools

import jax
import jax.numpy as jnp
from jax import lax
from jax.experimental import pallas as pl
from jax.experimental.pallas import tpu as pltpu
from jax.experimental.pallas import tpu_sc as plsc

_NC = 2
_NS = 16
_NW = _NC * _NS
_L = 16

_R = 128
_C = 8192
_K = 3
_R_SC = 32
_R_TC = _R - _R_SC
_NQ = 4
_QC = _C // _NQ
_NVQ = _QC // _L
_OPAD = 16


def _insert(state, v, iv):
  m1, m2, m3, i1, i2, i3 = state
  c1 = v > m1
  c2 = v > m2
  c3 = v > m3
  t1 = jnp.minimum(m1, v)
  n1 = jnp.maximum(m1, v)
  t2 = jnp.minimum(m2, t1)
  n2 = jnp.maximum(m2, t1)
  n3 = jnp.maximum(m3, t2)
  j1 = jnp.where(c1, iv, i1)
  j2 = jnp.where(c1, i1, jnp.where(c2, iv, i2))
  j3 = jnp.where(c2, i2, jnp.where(c3, iv, i3))
  return n1, n2, n3, j1, j2, j3


def _bcast_max(x):
  return plsc.cummax(jnp.flip(plsc.cummax(x)))


def _bcast_min_i32(x):
  return -_bcast_max(-x)


def _sc_body(x_hbm, out_hbm, xin, vout):
  w = lax.axis_index("s") * _NC + lax.axis_index("c")
  pltpu.sync_copy(x_hbm.at[pl.ds(w, 1)], xin)

  neg_inf = jnp.full((_L,), -jnp.inf, jnp.float32)
  zeros_i = jnp.zeros((_L,), jnp.int32)
  iota = lax.iota(jnp.int32, _L)
  big = jnp.full((_L,), jnp.int32(2**30))

  def body(t, carry):
    new = []
    iv = carry[-1]
    for q in range(_NQ):
      v = xin[0, pl.ds(q * _QC + t * _L, _L)]
      new.append(_insert(carry[q], v, iv + q * _QC))
    return (*new, iv + _L)

  init = tuple((neg_inf, neg_inf, neg_inf, zeros_i, zeros_i, zeros_i)
               for _ in range(_NQ))
  carry = lax.fori_loop(0, _NVQ, body, (*init, iota), unroll=2)

  st = carry[0]
  for q in range(1, _NQ):
    m1, m2, m3, i1, i2, i3 = carry[q]
    st = _insert(st, m1, i1)
    st = _insert(st, m2, i2)
    st = _insert(st, m3, i3)
  m1, m2, m3, i1, i2, i3 = st

  vacc = neg_inf
  iacc = zeros_i
  for k in range(_K):
    s = _bcast_max(m1)
    cand = _bcast_min_i32(jnp.where(m1 == s, i1, big))
    vacc = jnp.where(iota == k, s, vacc)
    iacc = jnp.where(iota == k, cand, iacc)
    hit = (m1 == s) & (i1 == cand)
    m1 = jnp.where(hit, m2, m1)
    i1 = jnp.where(hit, i2, i1)
    m2 = jnp.where(hit, m3, m2)
    i2 = jnp.where(hit, i3, i2)
    m3 = jnp.where(hit, neg_inf, m3)
  vout[0, 0, :] = vacc
  vout[1, 0, :] = plsc.bitcast(iacc, jnp.float32)

  pltpu.sync_copy(vout.at[0], out_hbm.at[0, pl.ds(w, 1)])
  pltpu.sync_copy(vout.at[1], out_hbm.at[1, pl.ds(w, 1)])


_topk_sc = functools.partial(
    pl.kernel,
    out_type=jax.ShapeDtypeStruct((2, _R_SC, _OPAD), jnp.float32),
    mesh=plsc.VectorSubcoreMesh(
        core_axis_name="c", subcore_axis_name="s",
        num_cores=_NC, num_subcores=_NS),
    compiler_params=pltpu.CompilerParams(needs_layout_passes=False),
    scratch_types=[
        pltpu.VMEM((1, _C), jnp.float32),
        pltpu.VMEM((2, 1, _OPAD), jnp.float32),
    ],
)(_sc_body)


_TB = 8
_TL = 128
_NVT = _C // _TL


_TG = 4
_NVG = _NVT // _TG


def _tc_kernel(x_ref, v_ref, i_ref):
  neg_inf = jnp.full((_TB, _TL), -jnp.inf, jnp.float32)
  zeros_i = jnp.zeros((_TB, _TL), jnp.int32)
  lane = lax.broadcasted_iota(jnp.int32, (_TB, _TL), 1)
  big = jnp.full((_TB, _TL), jnp.int32(2**30))

  carry = [
      (neg_inf, neg_inf, neg_inf, zeros_i, zeros_i, zeros_i)
      for _ in range(_TG)
  ]
  for t in range(_NVG):
    for g in range(_TG):
      c = g * _NVG + t
      v = x_ref[:, pl.ds(c * _TL, _TL)]
      carry[g] = _insert(carry[g], v, lane + c * _TL)

  st = carry[0]
  for g in range(1, _TG):
    m1, m2, m3, i1, i2, i3 = carry[g]
    st = _insert(st, m1, i1)
    st = _insert(st, m2, i2)
    st = _insert(st, m3, i3)
  m1, m2, m3, i1, i2, i3 = st

  vcols = []
  icols = []
  for _ in range(_K):
    s = jnp.max(m1, axis=1, keepdims=True)
    cand = jnp.min(jnp.where(m1 == s, i1, big), axis=1, keepdims=True)
    vcols.append(s)
    icols.append(cand)
    hit = (m1 == s) & (i1 == cand)
    m1 = jnp.where(hit, m2, m1)
    i1 = jnp.where(hit, i2, i1)
    m2 = jnp.where(hit, m3, m2)
    i2 = jnp.where(hit, i3, i2)
    m3 = jnp.where(hit, neg_inf, m3)
  v_ref[...] = jnp.concatenate(vcols, axis=1)
  i_ref[...] = jnp.concatenate(icols, axis=1)


_topk_tc = pl.pallas_call(
    _tc_kernel,
    grid=(_R_TC // _TB,),
    in_specs=[pl.BlockSpec((_TB, _C), lambda i: (i + _R_SC // _TB, 0))],
    out_specs=[
        pl.BlockSpec((_TB, _K), lambda i: (i, 0)),
        pl.BlockSpec((_TB, _K), lambda i: (i, 0)),
    ],
    out_shape=[
        jax.ShapeDtypeStruct((_R_TC, _K), jnp.float32),
        jax.ShapeDtypeStruct((_R_TC, _K), jnp.int32),
    ],
)


@jax.jit
def kernel(x):
  sc_out = _topk_sc(x)
  tc_vals, tc_idxs = _topk_tc(x)
  sc_sliced = sc_out[:, :, :_K]
  vals = jnp.concatenate([sc_sliced[0], tc_vals], axis=0)
  idxs = jnp.concatenate(
      [lax.bitcast_convert_type(sc_sliced[1], jnp.int32), tc_idxs], axis=0)
  return vals, idxs


# speedup vs baseline: 1.1468x; 1.1128x over previous
import functools

import jax
import jax.numpy as jnp
from jax import lax
from jax.experimental import pallas as pl
from jax.experimental.pallas import tpu as pltpu
from jax.experimental.pallas import tpu_sc as plsc

_NC = 2
_NS = 16
_NW = _NC * _NS
_L = 16

_R = 128
_C = 8192
_K = 3
_R_SC = 64
_RPW = _R_SC // _NW
_R_TC = _R - _R_SC
_NQ = 4
_QC = _C // _NQ
_NVQ = _QC // _L
_OPAD = 16


def _insert(state, v, iv):
  m1, m2, m3, i1, i2, i3 = state
  c1 = v > m1
  c2 = v > m2
  c3 = v > m3
  t1 = jnp.minimum(m1, v)
  n1 = jnp.maximum(m1, v)
  t2 = jnp.minimum(m2, t1)
  n2 = jnp.maximum(m2, t1)
  n3 = jnp.maximum(m3, t2)
  j1 = jnp.where(c1, iv, i1)
  j2 = jnp.where(c1, i1, jnp.where(c2, iv, i2))
  j3 = jnp.where(c2, i2, jnp.where(c3, iv, i3))
  return n1, n2, n3, j1, j2, j3


def _bcast_max(x):
  return plsc.cummax(jnp.flip(plsc.cummax(x)))


def _bcast_min_i32(x):
  return -_bcast_max(-x)


def _sc_body(x_hbm, out_hbm, xin, vout):
  w = lax.axis_index("s") * _NC + lax.axis_index("c")
  base = w * _RPW
  pltpu.sync_copy(x_hbm.at[pl.ds(base, _RPW)], xin)

  neg_inf = jnp.full((_L,), -jnp.inf, jnp.float32)
  zeros_i = jnp.zeros((_L,), jnp.int32)
  iota = lax.iota(jnp.int32, _L)
  big = jnp.full((_L,), jnp.int32(2**30))

  for r in range(_RPW):
    def body(t, carry):
      new = []
      iv = carry[-1]
      for q in range(_NQ):
        v = xin[r, pl.ds(q * _QC + t * _L, _L)]
        new.append(_insert(carry[q], v, iv + q * _QC))
      return (*new, iv + _L)

    init = tuple((neg_inf, neg_inf, neg_inf, zeros_i, zeros_i, zeros_i)
                 for _ in range(_NQ))
    carry = lax.fori_loop(0, _NVQ, body, (*init, iota), unroll=2)

    st = carry[0]
    for q in range(1, _NQ):
      m1, m2, m3, i1, i2, i3 = carry[q]
      st = _insert(st, m1, i1)
      st = _insert(st, m2, i2)
      st = _insert(st, m3, i3)
    m1, m2, m3, i1, i2, i3 = st

    vacc = neg_inf
    iacc = zeros_i
    for k in range(_K):
      s = _bcast_max(m1)
      cand = _bcast_min_i32(jnp.where(m1 == s, i1, big))
      vacc = jnp.where(iota == k, s, vacc)
      iacc = jnp.where(iota == k, cand, iacc)
      hit = (m1 == s) & (i1 == cand)
      m1 = jnp.where(hit, m2, m1)
      i1 = jnp.where(hit, i2, i1)
      m2 = jnp.where(hit, m3, m2)
      i2 = jnp.where(hit, i3, i2)
      m3 = jnp.where(hit, neg_inf, m3)
    vout[0, r, :] = vacc
    vout[1, r, :] = plsc.bitcast(iacc, jnp.float32)

  pltpu.sync_copy(vout.at[0], out_hbm.at[0, pl.ds(base, _RPW)])
  pltpu.sync_copy(vout.at[1], out_hbm.at[1, pl.ds(base, _RPW)])


_topk_sc = functools.partial(
    pl.kernel,
    out_type=jax.ShapeDtypeStruct((2, _R_SC, _OPAD), jnp.float32),
    mesh=plsc.VectorSubcoreMesh(
        core_axis_name="c", subcore_axis_name="s",
        num_cores=_NC, num_subcores=_NS),
    compiler_params=pltpu.CompilerParams(needs_layout_passes=False),
    scratch_types=[
        pltpu.VMEM((_RPW, _C), jnp.float32),
        pltpu.VMEM((2, _RPW, _OPAD), jnp.float32),
    ],
)(_sc_body)


_TB = 8
_TL = 128
_NVT = _C // _TL


_TG = 4
_NVG = _NVT // _TG


def _tc_kernel(x_ref, v_ref, i_ref):
  neg_inf = jnp.full((_TB, _TL), -jnp.inf, jnp.float32)
  zeros_i = jnp.zeros((_TB, _TL), jnp.int32)
  lane = lax.broadcasted_iota(jnp.int32, (_TB, _TL), 1)
  big = jnp.full((_TB, _TL), jnp.int32(2**30))

  carry = [
      (neg_inf, neg_inf, neg_inf, zeros_i, zeros_i, zeros_i)
      for _ in range(_TG)
  ]
  for t in range(_NVG):
    for g in range(_TG):
      c = g * _NVG + t
      v = x_ref[:, pl.ds(c * _TL, _TL)]
      carry[g] = _insert(carry[g], v, lane + c * _TL)

  st = carry[0]
  for g in range(1, _TG):
    m1, m2, m3, i1, i2, i3 = carry[g]
    st = _insert(st, m1, i1)
    st = _insert(st, m2, i2)
    st = _insert(st, m3, i3)
  m1, m2, m3, i1, i2, i3 = st

  vcols = []
  icols = []
  for _ in range(_K):
    s = jnp.max(m1, axis=1, keepdims=True)
    cand = jnp.min(jnp.where(m1 == s, i1, big), axis=1, keepdims=True)
    vcols.append(s)
    icols.append(cand)
    hit = (m1 == s) & (i1 == cand)
    m1 = jnp.where(hit, m2, m1)
    i1 = jnp.where(hit, i2, i1)
    m2 = jnp.where(hit, m3, m2)
    i2 = jnp.where(hit, i3, i2)
    m3 = jnp.where(hit, neg_inf, m3)
  v_ref[...] = jnp.concatenate(vcols, axis=1)
  i_ref[...] = jnp.concatenate(icols, axis=1)


_topk_tc = pl.pallas_call(
    _tc_kernel,
    grid=(_R_TC // _TB,),
    in_specs=[pl.BlockSpec((_TB, _C), lambda i: (i + _R_SC // _TB, 0))],
    out_specs=[
        pl.BlockSpec((_TB, _K), lambda i: (i, 0)),
        pl.BlockSpec((_TB, _K), lambda i: (i, 0)),
    ],
    out_shape=[
        jax.ShapeDtypeStruct((_R_TC, _K), jnp.float32),
        jax.ShapeDtypeStruct((_R_TC, _K), jnp.int32),
    ],
)


@jax.jit
def kernel(x):
  sc_out = _topk_sc(x)
  tc_vals, tc_idxs = _topk_tc(x)
  sc_sliced = sc_out[:, :, :_K]
  vals = jnp.concatenate([sc_sliced[0], tc_vals], axis=0)
  idxs = jnp.concatenate(
      [lax.bitcast_convert_type(sc_sliced[1], jnp.int32), tc_idxs], axis=0)
  return vals, idxs
